# baseline (speedup 1.0000x reference)
import functools

import jax
import jax.numpy as jnp
from jax import lax
from jax.experimental import pallas as pl
from jax.experimental.pallas import tpu as pltpu
from jax.experimental.pallas import tpu_sc as plsc

VOCAB = 100000
D = 2048
SEQ = 2048
BATCH = 4
ROWS = BATCH * SEQ
NC, NS, LANES = 2, 16, 16
NW = NC * NS
POS_PER_W = SEQ // NW
PGRP = 8
NGRP = POS_PER_W // PGRP
CHUNK = 8
CPG = BATCH
NCH = NGRP * CPG
NRB = 4


def _sc_embed(table, idx4d, pos2d):
    mesh = plsc.VectorSubcoreMesh(core_axis_name="c", subcore_axis_name="s")

    @functools.partial(
        pl.kernel,
        out_type=jax.ShapeDtypeStruct((ROWS, D), jnp.float32),
        mesh=mesh,
        scratch_types=[
            pltpu.VMEM((BATCH, POS_PER_W), jnp.int32),
            pltpu.VMEM((NRB, CHUNK, D), jnp.float32),
            pltpu.VMEM((2, PGRP, D), jnp.float32),
            [pltpu.SemaphoreType.DMA] * NRB,
            [pltpu.SemaphoreType.DMA] * NRB,
            [pltpu.SemaphoreType.DMA] * 2,
        ],
    )
    def k(table_hbm, idx_hbm, pos_hbm, out_hbm,
          idx_v, rb, pb, gsem, ssem, psem):
        wid = lax.axis_index("c") * NS + lax.axis_index("s")
        pos0 = wid * POS_PER_W

        def gather_descr(t, j):
            g, b = t // CPG, t % CPG
            return pltpu.make_async_copy(
                table_hbm.at[idx_v.at[b, pl.ds(g * PGRP, PGRP)]],
                rb.at[j], gsem[j])

        def store_descr(t, j):
            g, b = t // CPG, t % CPG
            out_base = b * SEQ + pos0 + g * PGRP
            return pltpu.make_async_copy(
                rb.at[j], out_hbm.at[pl.ds(out_base, CHUNK)], ssem[j])

        def pos_descr(g, pj):
            return pltpu.make_async_copy(
                pos_hbm.at[pl.ds(pos0 + g * PGRP, PGRP)], pb.at[pj], psem[pj])

        for b in range(BATCH):
            pltpu.sync_copy(idx_hbm.at[b, pl.ds(pos0, POS_PER_W)],
                            idx_v.at[b])
        pos_descr(0, 0).start()
        gather_descr(0, 0).start()
        gather_descr(1, 1).start()

        @pl.loop(0, NCH, step=2 * CPG)
        def _(tt):
            for jj in range(2 * CPG):
                t = tt + jj
                j = jj % NRB
                pg = (jj // CPG) % 2
                g = t // CPG

                if jj % CPG == 0:
                    pos_descr(g, pg).wait()

                    @pl.when(g < NGRP - 1)
                    def _():
                        pos_descr(g + 1, 1 - pg).start()

                gather_descr(t, j).wait()

                @pl.when(t + 2 < NCH)
                def _():
                    j2 = (jj + 2) % NRB

                    @pl.when(t >= 2)
                    def _():
                        store_descr(t - 2, j2).wait()

                    gather_descr(t + 2, j2).start()

                @pl.loop(0, CHUNK)
                def _(r):
                    @pl.loop(0, D, step=LANES, unroll=16)
                    def _(col):
                        rb.at[j, r, pl.ds(col, LANES)][...] = (
                            rb.at[j, r, pl.ds(col, LANES)][...]
                            + pb.at[pg, r, pl.ds(col, LANES)][...]
                        )

                store_descr(t, j).start()

        for jj in range(NRB):
            t = NCH - NRB + jj
            store_descr(t, t % NRB).wait()

    return k(table, idx4d, pos2d)


def kernel(x, table, pos_embed):
    pos2d = pos_embed.reshape(SEQ, D)
    out = _sc_embed(table, x.astype(jnp.int32), pos2d)
    return out.reshape(BATCH, SEQ, D)


# speedup vs baseline: 1.1525x; 1.1525x over previous
import functools

import jax
import jax.numpy as jnp
from jax import lax
from jax.experimental import pallas as pl
from jax.experimental.pallas import tpu as pltpu
from jax.experimental.pallas import tpu_sc as plsc

VOCAB = 100000
D = 2048
SEQ = 2048
BATCH = 4
ROWS = BATCH * SEQ
NC, NS, LANES = 2, 16, 16
NW = NC * NS
POS_PER_W = SEQ // NW
PGRP = 8
NGRP = POS_PER_W // PGRP
CHUNK = 8
CPG = BATCH
NCH = NGRP * CPG
NRB = 4


def _sc_embed(table, idx4d, pos2d):
    mesh = plsc.VectorSubcoreMesh(core_axis_name="c", subcore_axis_name="s")

    @functools.partial(
        pl.kernel,
        out_type=jax.ShapeDtypeStruct((ROWS, D), jnp.float32),
        mesh=mesh,
        scratch_types=[
            pltpu.VMEM((BATCH, POS_PER_W), jnp.int32),
            pltpu.VMEM((NRB, CHUNK, D), jnp.float32),
            pltpu.VMEM((2, PGRP, D), jnp.float32),
            [pltpu.SemaphoreType.DMA] * NRB,
            [pltpu.SemaphoreType.DMA] * NRB,
            [pltpu.SemaphoreType.DMA] * 2,
        ],
    )
    def k(table_hbm, idx_hbm, pos_hbm, out_hbm,
          idx_v, rb, pb, gsem, ssem, psem):
        wid = lax.axis_index("c") * NS + lax.axis_index("s")
        pos0 = wid * POS_PER_W

        def gather_descr(t, j):
            g, b = t // CPG, t % CPG
            return pltpu.make_async_copy(
                table_hbm.at[idx_v.at[b, pl.ds(g * PGRP, PGRP)]],
                rb.at[j], gsem[j])

        def store_descr(t, j):
            g, b = t // CPG, t % CPG
            out_base = b * SEQ + pos0 + g * PGRP
            return pltpu.make_async_copy(
                rb.at[j], out_hbm.at[pl.ds(out_base, CHUNK)], ssem[j])

        def pos_descr(g, pj):
            return pltpu.make_async_copy(
                pos_hbm.at[pl.ds(pos0 + g * PGRP, PGRP)], pb.at[pj], psem[pj])

        idx_cp = [pltpu.make_async_copy(
            idx_hbm.at[b, pl.ds(pos0, POS_PER_W)], idx_v.at[b], psem[0])
            for b in range(BATCH)]
        for c in idx_cp:
            c.start()
        for c in idx_cp:
            c.wait()
        pos_descr(0, 0).start()
        gather_descr(0, 0).start()
        gather_descr(1, 1).start()

        @pl.loop(0, NCH, step=2 * CPG)
        def _(tt):
            for jj in range(2 * CPG):
                t = tt + jj
                j = jj % NRB
                pg = (jj // CPG) % 2
                g = t // CPG

                if jj % CPG == 0:
                    pos_descr(g, pg).wait()

                    @pl.when(g < NGRP - 1)
                    def _():
                        pos_descr(g + 1, 1 - pg).start()

                gather_descr(t, j).wait()

                @pl.when(t + 2 < NCH)
                def _():
                    j2 = (jj + 2) % NRB

                    @pl.when(t >= 2)
                    def _():
                        store_descr(t - 2, j2).wait()

                    gather_descr(t + 2, j2).start()

                @pl.loop(0, CHUNK)
                def _(r):
                    @pl.loop(0, D, step=LANES, unroll=16)
                    def _(col):
                        plsc.addupdate(
                            rb.at[j, r, pl.ds(col, LANES)],
                            pb.at[pg, r, pl.ds(col, LANES)][...])

                store_descr(t, j).start()

        for jj in range(NRB):
            t = NCH - NRB + jj
            store_descr(t, t % NRB).wait()

    return k(table, idx4d, pos2d)


def kernel(x, table, pos_embed):
    pos2d = pos_embed.reshape(SEQ, D)
    out = _sc_embed(table, x.astype(jnp.int32), pos2d)
    return out.reshape(BATCH, SEQ, D)


# speedup vs baseline: 1.1525x; 1.0000x over previous
import functools

import jax
import jax.numpy as jnp
from jax import lax
from jax.experimental import pallas as pl
from jax.experimental.pallas import tpu as pltpu
from jax.experimental.pallas import tpu_sc as plsc

VOCAB = 100000
D = 2048
SEQ = 2048
BATCH = 4
ROWS = BATCH * SEQ
NC, NS, LANES = 2, 16, 16
NW = NC * NS
POS_PER_W = SEQ // NW
PGRP = 8
NGRP = POS_PER_W // PGRP
CHUNK = 8
CPG = BATCH
NCH = NGRP * CPG
NRB = 4


def _sc_embed(table, idx4d, pos2d):
    mesh = plsc.VectorSubcoreMesh(core_axis_name="c", subcore_axis_name="s")

    @functools.partial(
        pl.kernel,
        out_type=jax.ShapeDtypeStruct((ROWS, D), jnp.float32),
        mesh=mesh,
        scratch_types=[
            pltpu.VMEM((BATCH, POS_PER_W), jnp.int32),
            pltpu.VMEM((NRB, CHUNK, D), jnp.float32),
            pltpu.VMEM((2, PGRP, D), jnp.float32),
            [pltpu.SemaphoreType.DMA] * NRB,
            [pltpu.SemaphoreType.DMA] * NRB,
            [pltpu.SemaphoreType.DMA] * 2,
        ],
    )
    def k(table_hbm, idx_hbm, pos_hbm, out_hbm,
          idx_v, rb, pb, gsem, ssem, psem):
        wid = lax.axis_index("c") * NS + lax.axis_index("s")
        pos0 = wid * POS_PER_W

        def gather_descr(t, j):
            g, b = t // CPG, t % CPG
            return pltpu.make_async_copy(
                table_hbm.at[idx_v.at[b, pl.ds(g * PGRP, PGRP)]],
                rb.at[j], gsem[j])

        def store_descr(t, j):
            g, b = t // CPG, t % CPG
            out_base = b * SEQ + pos0 + g * PGRP
            return pltpu.make_async_copy(
                rb.at[j], out_hbm.at[pl.ds(out_base, CHUNK)], ssem[j])

        def pos_descr(g, pj):
            return pltpu.make_async_copy(
                pos_hbm.at[pl.ds(pos0 + g * PGRP, PGRP)], pb.at[pj], psem[pj])

        idx_cp = [pltpu.make_async_copy(
            idx_hbm.at[b, pl.ds(pos0, POS_PER_W)], idx_v.at[b], psem[0])
            for b in range(BATCH)]
        for c in idx_cp:
            c.start()
        for c in idx_cp:
            c.wait()
        pos_descr(0, 0).start()
        gather_descr(0, 0).start()
        gather_descr(1, 1).start()

        @pl.loop(0, NCH, step=2 * CPG)
        def _(tt):
            for jj in range(2 * CPG):
                t = tt + jj
                j = jj % NRB
                pg = (jj // CPG) % 2
                g = t // CPG

                if jj % CPG == 0:
                    pos_descr(g, pg).wait()

                    @pl.when(g < NGRP - 1)
                    def _():
                        pos_descr(g + 1, 1 - pg).start()

                gather_descr(t, j).wait()

                @pl.when(t + 2 < NCH)
                def _():
                    j2 = (jj + 2) % NRB

                    @pl.when(t >= 2)
                    def _():
                        store_descr(t - 2, j2).wait()

                    gather_descr(t + 2, j2).start()

                @plsc.parallel_loop(0, CHUNK)
                def _(r):
                    @plsc.parallel_loop(0, D, step=LANES, unroll=16)
                    def _(col):
                        plsc.addupdate(
                            rb.at[j, r, pl.ds(col, LANES)],
                            pb.at[pg, r, pl.ds(col, LANES)][...])

                store_descr(t, j).start()

        for jj in range(NRB):
            t = NCH - NRB + jj
            store_descr(t, t % NRB).wait()

    return k(table, idx4d, pos2d)


def kernel(x, table, pos_embed):
    pos2d = pos_embed.reshape(SEQ, D)
    out = _sc_embed(table, x.astype(jnp.int32), pos2d)
    return out.reshape(BATCH, SEQ, D)


# speedup vs baseline: 1.1663x; 1.0120x over previous
import functools

import jax
import jax.numpy as jnp
from jax import lax
from jax.experimental import pallas as pl
from jax.experimental.pallas import tpu as pltpu
from jax.experimental.pallas import tpu_sc as plsc

VOCAB = 100000
D = 2048
SEQ = 2048
BATCH = 4
ROWS = BATCH * SEQ
NC, NS, LANES = 2, 16, 16
NW = NC * NS
POS_PER_W = SEQ // NW
PGRP = 8
NGRP = POS_PER_W // PGRP
CHUNK = 8
CPG = BATCH
NCH = NGRP * CPG
NRB = 4


def _sc_embed(table, idx4d, pos2d):
    mesh = plsc.VectorSubcoreMesh(core_axis_name="c", subcore_axis_name="s")

    @functools.partial(
        pl.kernel,
        out_type=jax.ShapeDtypeStruct((ROWS, D), jnp.float32),
        mesh=mesh,
        scratch_types=[
            pltpu.VMEM((BATCH, POS_PER_W), jnp.int32),
            pltpu.VMEM((NRB, CHUNK, D), jnp.float32),
            pltpu.VMEM((2, PGRP, D), jnp.float32),
            [pltpu.SemaphoreType.DMA] * NRB,
            [pltpu.SemaphoreType.DMA] * NRB,
            [pltpu.SemaphoreType.DMA] * 2,
        ],
    )
    def k(table_hbm, idx_hbm, pos_hbm, out_hbm,
          idx_v, rb, pb, gsem, ssem, psem):
        wid = lax.axis_index("c") * NS + lax.axis_index("s")
        pos0 = wid * POS_PER_W

        def gather_descr(t, j):
            g, b = t // CPG, t % CPG
            return pltpu.make_async_copy(
                table_hbm.at[idx_v.at[b, pl.ds(g * PGRP, PGRP)]],
                rb.at[j], gsem[j])

        def store_descr(t, j):
            g, b = t // CPG, t % CPG
            out_base = b * SEQ + pos0 + g * PGRP
            return pltpu.make_async_copy(
                rb.at[j], out_hbm.at[pl.ds(out_base, CHUNK)], ssem[j])

        def pos_descr(g, pj):
            return pltpu.make_async_copy(
                pos_hbm.at[pl.ds(pos0 + g * PGRP, PGRP)], pb.at[pj], psem[pj])

        pos_descr(0, 0).start()
        idx_cp = [pltpu.make_async_copy(
            idx_hbm.at[b, pl.ds(pos0, POS_PER_W)], idx_v.at[b], ssem[b])
            for b in range(BATCH)]
        for c in idx_cp:
            c.start()
        idx_cp[0].wait()
        gather_descr(0, 0).start()
        idx_cp[1].wait()
        gather_descr(1, 1).start()
        idx_cp[2].wait()
        idx_cp[3].wait()

        @pl.loop(0, NCH, step=2 * CPG)
        def _(tt):
            for jj in range(2 * CPG):
                t = tt + jj
                j = jj % NRB
                pg = (jj // CPG) % 2
                g = t // CPG

                if jj % CPG == 0:
                    pos_descr(g, pg).wait()

                    @pl.when(g < NGRP - 1)
                    def _():
                        pos_descr(g + 1, 1 - pg).start()

                gather_descr(t, j).wait()

                @pl.when(t + 2 < NCH)
                def _():
                    j2 = (jj + 2) % NRB

                    @pl.when(t >= 2)
                    def _():
                        store_descr(t - 2, j2).wait()

                    gather_descr(t + 2, j2).start()

                @plsc.parallel_loop(0, CHUNK)
                def _(r):
                    @plsc.parallel_loop(0, D, step=LANES, unroll=16)
                    def _(col):
                        plsc.addupdate(
                            rb.at[j, r, pl.ds(col, LANES)],
                            pb.at[pg, r, pl.ds(col, LANES)][...])

                store_descr(t, j).start()

        for jj in range(NRB):
            t = NCH - NRB + jj
            store_descr(t, t % NRB).wait()

    return k(table, idx4d, pos2d)


def kernel(x, table, pos_embed):
    pos2d = pos_embed.reshape(SEQ, D)
    out = _sc_embed(table, x.astype(jnp.int32), pos2d)
    return out.reshape(BATCH, SEQ, D)
